# Initial kernel scaffold; baseline (speedup 1.0000x reference)
import jax
import jax.numpy as jnp
from jax.experimental import pallas as pl


def kernel(x):
    raise NotImplementedError("write your pallas kernel here")


# speedup vs baseline: 1.3527x; 1.3527x over previous
import functools

import jax
import jax.numpy as jnp
import numpy as np
from jax.experimental import pallas as pl
from jax.experimental.pallas import tpu as pltpu


def _pairs():
    arg1s = [[8, 9], [17, 18], [26, 27]]
    arg2s = [[11, 12, 13, 14, 15, 16], [20, 21, 22, 23, 24, 25],
             [29, 30, 31, 32, 33, 34]]
    prods = []
    for a, b in zip(arg1s, arg2s):
        for i in a:
            for j in b:
                prods.append((i, j))
    return np.array(prods, dtype=np.int32)


_P = _pairs()
_S0 = np.zeros((64, 36), np.float32)
_S0[_P[:, 0], np.arange(36)] = 1.0
_S1 = np.zeros((64, 36), np.float32)
_S1[_P[:, 1], np.arange(36)] = 1.0

_ROWS = 2048 * 512
_BLK = 4096


def _body(x_ref, s0_ref, s1_ref, o_ref):
    xb = x_ref[...]
    a = jnp.dot(xb, s0_ref[...], preferred_element_type=jnp.float32)
    b = jnp.dot(xb, s1_ref[...], preferred_element_type=jnp.float32)
    o_ref[...] = jnp.concatenate([xb, a * b], axis=-1)


@jax.jit
def kernel(x):
    xf = x.reshape(_ROWS, 64)
    out = pl.pallas_call(
        _body,
        grid=(_ROWS // _BLK,),
        in_specs=[
            pl.BlockSpec((_BLK, 64), lambda i: (i, 0)),
            pl.BlockSpec((64, 36), lambda i: (0, 0)),
            pl.BlockSpec((64, 36), lambda i: (0, 0)),
        ],
        out_specs=pl.BlockSpec((_BLK, 100), lambda i: (i, 0)),
        out_shape=jax.ShapeDtypeStruct((_ROWS, 100), jnp.float32),
    )(xf, jnp.asarray(_S0), jnp.asarray(_S1))
    return out.reshape(x.shape[0], x.shape[1], 100)


# speedup vs baseline: 1.4906x; 1.1019x over previous
import functools

import jax
import jax.numpy as jnp
import numpy as np
from jax.experimental import pallas as pl
from jax.experimental.pallas import tpu as pltpu


def _pairs():
    arg1s = [[8, 9], [17, 18], [26, 27]]
    arg2s = [[11, 12, 13, 14, 15, 16], [20, 21, 22, 23, 24, 25],
             [29, 30, 31, 32, 33, 34]]
    prods = []
    for a, b in zip(arg1s, arg2s):
        for i in a:
            for j in b:
                prods.append((i, j))
    return np.array(prods, dtype=np.int32)


_P = _pairs()
_S0 = np.zeros((64, 36), np.float32)
_S0[_P[:, 0], np.arange(36)] = 1.0
_S1 = np.zeros((64, 36), np.float32)
_S1[_P[:, 1], np.arange(36)] = 1.0

_ROWS = 2048 * 512
_BLK = 16384


def _body(x_ref, s0_ref, s1_ref, o_ref):
    xb = x_ref[...]
    a = jnp.dot(xb, s0_ref[...], preferred_element_type=jnp.float32)
    b = jnp.dot(xb, s1_ref[...], preferred_element_type=jnp.float32)
    o_ref[...] = jnp.concatenate([xb, a * b], axis=-1)


@jax.jit
def kernel(x):
    xf = x.reshape(_ROWS, 64)
    out = pl.pallas_call(
        _body,
        grid=(_ROWS // _BLK,),
        in_specs=[
            pl.BlockSpec((_BLK, 64), lambda i: (i, 0)),
            pl.BlockSpec((64, 36), lambda i: (0, 0)),
            pl.BlockSpec((64, 36), lambda i: (0, 0)),
        ],
        out_specs=pl.BlockSpec((_BLK, 100), lambda i: (i, 0)),
        out_shape=jax.ShapeDtypeStruct((_ROWS, 100), jnp.float32),
    )(xf, jnp.asarray(_S0), jnp.asarray(_S1))
    return out.reshape(x.shape[0], x.shape[1], 100)
